# baseline (speedup 1.0000x reference)
import functools

import jax
import jax.numpy as jnp
from jax import lax
from jax.experimental import pallas as pl
from jax.experimental.pallas import tpu as pltpu
from jax.experimental.pallas import tpu_sc as plsc

_PADDING_IDX = 1
_NC = 2
_NS = 16
_NW = _NC * _NS
_LANES = 16
_PC = 4
_NBUF = 6


def _bcast(x, n=_LANES):
    return lax.broadcast_in_dim(jnp.int32(x), (n,), ())


@functools.cache
def _build(seq_len: int, bsz: int, vocab: int, dim: int):
    B = seq_len * bsz
    ppw = seq_len // _NW
    rpw = B // _NW
    n_chunks = ppw // _PC
    cr = _PC * bsz
    assert ppw * _NW == seq_len and n_chunks * _PC == ppw
    assert rpw % _LANES == 0 and n_chunks >= _NBUF
    mesh = plsc.VectorSubcoreMesh(core_axis_name="c", subcore_axis_name="s",
                                  num_cores=_NC, num_subcores=_NS)

    @functools.partial(
        pl.kernel,
        out_type=jax.ShapeDtypeStruct((seq_len, bsz, dim), jnp.float32),
        mesh=mesh,
        compiler_params=pltpu.CompilerParams(needs_layout_passes=False),
        scratch_types=[
            pltpu.VMEM((ppw, bsz), jnp.int32),
            pltpu.VMEM((rpw,), jnp.int32),
            pltpu.VMEM((_NBUF, _PC, bsz, dim), jnp.float32),
            pltpu.SemaphoreType.DMA,
            pltpu.SemaphoreType.DMA,
        ],
    )
    def k(tok_hbm, w_hbm, out_hbm, tok_v, idx_v, wbuf, gsem, wsem):
        wid = lax.axis_index("s") * _NC + lax.axis_index("c")
        p0 = wid * ppw
        r0 = p0 * bsz
        iota = lax.iota(jnp.int32, _LANES)

        pltpu.sync_copy(tok_hbm.at[pl.ds(p0, ppw)], tok_v)
        sh = bsz.bit_length() - 1
        for i in range(rpw // _LANES):
            j = _bcast(i * _LANES) + iota
            if bsz & (bsz - 1) == 0:
                t = plsc.load_gather(tok_v, [j >> sh, j & (bsz - 1)])
            else:
                t = plsc.load_gather(tok_v, [j // bsz, j % bsz])
            g = _bcast(r0 + i * _LANES) + iota
            gpos = (g >> bsz.bit_length() - 1 if bsz & (bsz - 1) == 0
                    else g // bsz) + (_PADDING_IDX + 1)
            idx_v[pl.ds(i * _LANES, _LANES)] = jnp.where(t != _PADDING_IDX,
                                                         gpos, t)

        def gather(c, slot):
            return pltpu.async_copy(
                w_hbm.at[idx_v.at[pl.ds(c * cr, cr)]],
                wbuf.at[slot].reshape(cr, dim), gsem)

        cp_g = [None] * _NBUF
        cp_w = [None] * _NBUF
        _LOOK = _NBUF - 1
        for c in range(_LOOK):
            cp_g[c] = gather(c, c)
        for c in range(n_chunks):
            slot = c % _NBUF
            cp_g[slot].wait()
            cp_w[slot] = pltpu.async_copy(
                wbuf.at[slot], out_hbm.at[pl.ds(p0 + c * _PC, _PC)], wsem)
            if c + _LOOK < n_chunks:
                s2 = (c + _LOOK) % _NBUF
                if cp_w[s2] is not None:
                    cp_w[s2].wait()
                cp_g[s2] = gather(c + _LOOK, s2)
        for cp in cp_w:
            if cp is not None:
                cp.wait()

    return k


def kernel(input, weights):
    seq_len, bsz = input.shape
    vocab, dim = weights.shape
    k = _build(seq_len, bsz, vocab, dim)
    return k(input, weights)


# speedup vs baseline: 1.0240x; 1.0240x over previous
import functools

import jax
import jax.numpy as jnp
from jax import lax
from jax.experimental import pallas as pl
from jax.experimental.pallas import tpu as pltpu
from jax.experimental.pallas import tpu_sc as plsc

_PADDING_IDX = 1
_NC = 2
_NS = 16
_NW = _NC * _NS
_LANES = 16
_NBUF = 6


def _bcast(x, n=_LANES):
    return lax.broadcast_in_dim(jnp.int32(x), (n,), ())


@functools.cache
def _build(seq_len: int, bsz: int, vocab: int, dim: int):
    B = seq_len * bsz
    ppw = seq_len // _NW
    rpw = B // _NW
    pc = _LANES // bsz
    n_chunks = ppw // pc
    cr = pc * bsz
    assert ppw * _NW == seq_len and n_chunks * pc == ppw
    assert pc * bsz == _LANES and n_chunks >= _NBUF
    mesh = plsc.VectorSubcoreMesh(core_axis_name="c", subcore_axis_name="s",
                                  num_cores=_NC, num_subcores=_NS)

    @functools.partial(
        pl.kernel,
        out_type=jax.ShapeDtypeStruct((seq_len, bsz, dim), jnp.float32),
        mesh=mesh,
        compiler_params=pltpu.CompilerParams(needs_layout_passes=False),
        scratch_types=[
            pltpu.VMEM((rpw,), jnp.int32),
            pltpu.VMEM((_NBUF, pc, bsz, dim), jnp.float32),
            pltpu.SemaphoreType.DMA,
            pltpu.SemaphoreType.DMA,
        ],
    )
    def k(tok_hbm, w_hbm, out_hbm, tok_v, wbuf, gsem, wsem):
        wid = lax.axis_index("s") * _NC + lax.axis_index("c")
        p0 = wid * ppw
        r0 = p0 * bsz
        iota = lax.iota(jnp.int32, _LANES)

        pltpu.sync_copy(tok_hbm.at[pl.ds(r0, rpw)], tok_v)

        def gather(c, slot):
            t = tok_v[pl.ds(c * cr, cr)]
            g = _bcast(r0 + c * cr) + iota
            gpos = (g >> bsz.bit_length() - 1 if bsz & (bsz - 1) == 0
                    else g // bsz) + (_PADDING_IDX + 1)
            idx = jnp.where(t != _PADDING_IDX, gpos, t)
            return pltpu.async_copy(
                w_hbm.at[idx], wbuf.at[slot].reshape(cr, dim), gsem)

        cp_g = [None] * _NBUF
        cp_w = [None] * _NBUF
        _LOOK = _NBUF - 1
        for c in range(_LOOK):
            cp_g[c] = gather(c, c)
        for c in range(n_chunks):
            slot = c % _NBUF
            cp_g[slot].wait()
            cp_w[slot] = pltpu.async_copy(
                wbuf.at[slot], out_hbm.at[pl.ds(p0 + c * pc, pc)], wsem)
            if c + _LOOK < n_chunks:
                s2 = (c + _LOOK) % _NBUF
                if cp_w[s2] is not None:
                    cp_w[s2].wait()
                cp_g[s2] = gather(c + _LOOK, s2)
        for cp in cp_w:
            if cp is not None:
                cp.wait()

    return k


def kernel(input, weights):
    seq_len, bsz = input.shape
    vocab, dim = weights.shape
    k = _build(seq_len, bsz, vocab, dim)
    return k(input.reshape(-1), weights)
